# baseline (speedup 1.0000x reference)
import functools

import jax
import jax.numpy as jnp
from jax import lax
from jax.experimental import pallas as pl
from jax.experimental.pallas import tpu as pltpu
from jax.experimental.pallas import tpu_sc as plsc

NC = 2
NS = 16
NW = NC * NS
LANES = 16
SCHUNK = 8
HCHUNK = 2048
NSLOT = 3
PSLOT = 2


def _make_sc_add(B, S, H):
    pos_per_w = S // NW
    n_sc = pos_per_w // SCHUNK
    n_hc = H // HCHUNK
    n_grp = n_sc * n_hc
    n_tiles = n_grp * B
    n_vec = HCHUNK // LANES

    mesh = plsc.VectorSubcoreMesh(core_axis_name="c", subcore_axis_name="s")

    @functools.partial(
        pl.kernel,
        mesh=mesh,
        out_type=jax.ShapeDtypeStruct((B, S, H), jnp.float32),
        scratch_types=(
            [pltpu.VMEM((SCHUNK, HCHUNK), jnp.float32)] * PSLOT
            + [pltpu.VMEM((SCHUNK, HCHUNK), jnp.float32)] * NSLOT
            + [pltpu.SemaphoreType.DMA] * (PSLOT + 2 * NSLOT)
        ),
        compiler_params=pltpu.CompilerParams(use_tc_tiling_on_sc=True),
    )
    def sc_add(x_hbm, p_hbm, o_hbm, *bufs):
        pos_v = bufs[0:PSLOT]
        buf_v = bufs[PSLOT:PSLOT + NSLOT]
        psem = bufs[PSLOT + NSLOT:2 * PSLOT + NSLOT]
        isem = bufs[2 * PSLOT + NSLOT:2 * PSLOT + 2 * NSLOT]
        osem = bufs[2 * PSLOT + 2 * NSLOT:]
        wid = lax.axis_index("s") * NC + lax.axis_index("c")
        s_base = wid * pos_per_w

        def grp_slices(g):
            c, hi = divmod(g, n_hc)
            return pl.ds(s_base + c * SCHUNK, SCHUNK), pl.ds(hi * HCHUNK, HCHUNK)

        def tile_slices(t):
            ssl, hsl = grp_slices(t // B)
            return t % B, ssl, hsl

        def start_pos(g):
            ssl, hsl = grp_slices(g)
            sl = g % PSLOT
            return pltpu.async_copy(p_hbm.at[ssl, hsl], pos_v[sl], psem[sl])

        def start_in(t):
            b, ssl, hsl = tile_slices(t)
            sl = t % NSLOT
            return pltpu.async_copy(x_hbm.at[b, ssl, hsl], buf_v[sl], isem[sl])

        def start_out(t):
            b, ssl, hsl = tile_slices(t)
            sl = t % NSLOT
            return pltpu.async_copy(buf_v[sl], o_hbm.at[b, ssl, hsl], osem[sl])

        def compute(sl, pp):
            @plsc.parallel_loop(0, n_vec * SCHUNK, unroll=2)
            def body(i):
                s = i & (SCHUNK - 1)
                j = i >> 3
                sli = pl.ds(j * LANES, LANES)
                buf_v[sl][s, sli] = buf_v[sl][s, sli] + pos_v[pp][s, sli]

        pos_h = {0: start_pos(0)}
        ins = {0: start_in(0), 1: start_in(1)}
        outs = {}
        for t in range(n_tiles):
            g = t // B
            if t % B == 0:
                pos_h.pop(g).wait()
            ins.pop(t).wait()
            compute(t % NSLOT, g % PSLOT)
            outs[t] = start_out(t)
            if t % B == 1 and g + 1 < n_grp:
                pos_h[g + 1] = start_pos(g + 1)
            if t + 2 < n_tiles:
                if t >= 1:
                    outs.pop(t - 1).wait()
                ins[t + 2] = start_in(t + 2)
        for t in sorted(outs):
            outs.pop(t).wait()

    return sc_add


def kernel(concat_embeddings, pos_table):
    B, S, H = concat_embeddings.shape
    sc_add = _make_sc_add(B, S, H)
    return sc_add(concat_embeddings, pos_table)


# speedup vs baseline: 1.1740x; 1.1740x over previous
import functools

import jax
import jax.numpy as jnp
from jax import lax
from jax.experimental import pallas as pl
from jax.experimental.pallas import tpu as pltpu
from jax.experimental.pallas import tpu_sc as plsc

NC = 2
NS = 16
NW = NC * NS
LANES = 16
SCHUNK = 8
HCHUNK = 512
NSLOT = 5


def _make_sc_add(B, S, H):
    pos_per_w = S // NW
    n_sc = pos_per_w // SCHUNK
    n_hc = H // HCHUNK
    n_tiles = n_sc * n_hc
    n_vec = HCHUNK // LANES

    mesh = plsc.VectorSubcoreMesh(core_axis_name="c", subcore_axis_name="s")

    @functools.partial(
        pl.kernel,
        mesh=mesh,
        out_type=jax.ShapeDtypeStruct((B, S, H), jnp.float32),
        scratch_types=(
            [pltpu.VMEM((SCHUNK, HCHUNK), jnp.float32)] * NSLOT
            + [pltpu.VMEM((B, SCHUNK, HCHUNK), jnp.float32)] * NSLOT
            + [pltpu.SemaphoreType.DMA] * (3 * NSLOT)
        ),
        compiler_params=pltpu.CompilerParams(use_tc_tiling_on_sc=True),
    )
    def sc_add(x_hbm, p_hbm, o_hbm, *bufs):
        pos_v = bufs[0:NSLOT]
        buf_v = bufs[NSLOT:2 * NSLOT]
        psem = bufs[2 * NSLOT:3 * NSLOT]
        isem = bufs[3 * NSLOT:4 * NSLOT]
        osem = bufs[4 * NSLOT:5 * NSLOT]
        wid = lax.axis_index("s") * NC + lax.axis_index("c")
        s_base = wid * pos_per_w

        def tile_slices(t):
            c, hi = divmod(t, n_hc)
            s0 = s_base + c * SCHUNK
            return pl.ds(s0, SCHUNK), pl.ds(hi * HCHUNK, HCHUNK)

        def start_in(t):
            sl = t % NSLOT
            ssl, hsl = tile_slices(t)
            return [
                pltpu.async_copy(p_hbm.at[ssl, hsl], pos_v[sl], psem[sl]),
                pltpu.async_copy(x_hbm.at[:, ssl, hsl], buf_v[sl], isem[sl]),
            ]

        def start_out(t):
            sl = t % NSLOT
            ssl, hsl = tile_slices(t)
            return [pltpu.async_copy(
                buf_v[sl], o_hbm.at[:, ssl, hsl], osem[sl])]

        def compute(sl):
            @plsc.parallel_loop(0, n_vec * SCHUNK, unroll=2)
            def body(i):
                s = i & (SCHUNK - 1)
                j = i >> 3
                sli = pl.ds(j * LANES, LANES)
                pv = pos_v[sl][s, sli]
                for b in range(B):
                    buf_v[sl][b, s, sli] = buf_v[sl][b, s, sli] + pv

        ins = {0: start_in(0), 1: start_in(1)}
        outs = {}
        for t in range(n_tiles):
            for h in ins.pop(t):
                h.wait()
            compute(t % NSLOT)
            outs[t] = start_out(t)
            if t + 2 < n_tiles:
                if t >= 1:
                    for h in outs.pop(t - 1):
                        h.wait()
                ins[t + 2] = start_in(t + 2)
        for t in sorted(outs):
            for h in outs.pop(t):
                h.wait()

    return sc_add


def kernel(concat_embeddings, pos_table):
    B, S, H = concat_embeddings.shape
    sc_add = _make_sc_add(B, S, H)
    return sc_add(concat_embeddings, pos_table)


# speedup vs baseline: 1.2552x; 1.0692x over previous
import functools

import jax
import jax.numpy as jnp
from jax import lax
from jax.experimental import pallas as pl
from jax.experimental.pallas import tpu as pltpu
from jax.experimental.pallas import tpu_sc as plsc

NC = 2
NS = 16
NW = NC * NS
LANES = 16
SCHUNK = 8
HCHUNK = 1024
NSLOT = 3


def _make_sc_add(B, S, H):
    pos_per_w = S // NW
    n_sc = pos_per_w // SCHUNK
    n_hc = H // HCHUNK
    n_tiles = n_sc * n_hc
    n_vec = HCHUNK // LANES

    mesh = plsc.VectorSubcoreMesh(core_axis_name="c", subcore_axis_name="s")

    @functools.partial(
        pl.kernel,
        mesh=mesh,
        out_type=jax.ShapeDtypeStruct((B, S, H), jnp.float32),
        scratch_types=(
            [pltpu.VMEM((SCHUNK, HCHUNK), jnp.float32)] * NSLOT
            + [pltpu.VMEM((B, SCHUNK, HCHUNK), jnp.float32)] * NSLOT
            + [pltpu.SemaphoreType.DMA] * (3 * NSLOT)
        ),
        compiler_params=pltpu.CompilerParams(use_tc_tiling_on_sc=True),
    )
    def sc_add(x_hbm, p_hbm, o_hbm, *bufs):
        pos_v = bufs[0:NSLOT]
        buf_v = bufs[NSLOT:2 * NSLOT]
        psem = bufs[2 * NSLOT:3 * NSLOT]
        isem = bufs[3 * NSLOT:4 * NSLOT]
        osem = bufs[4 * NSLOT:5 * NSLOT]
        wid = lax.axis_index("s") * NC + lax.axis_index("c")
        s_base = wid * pos_per_w

        def tile_slices(t):
            c, hi = divmod(t, n_hc)
            s0 = s_base + c * SCHUNK
            return pl.ds(s0, SCHUNK), pl.ds(hi * HCHUNK, HCHUNK)

        def start_in(t):
            sl = t % NSLOT
            ssl, hsl = tile_slices(t)
            return [
                pltpu.async_copy(p_hbm.at[ssl, hsl], pos_v[sl], psem[sl]),
                pltpu.async_copy(x_hbm.at[:, ssl, hsl], buf_v[sl], isem[sl]),
            ]

        def start_out(t):
            sl = t % NSLOT
            ssl, hsl = tile_slices(t)
            return [pltpu.async_copy(
                buf_v[sl], o_hbm.at[:, ssl, hsl], osem[sl])]

        def compute(sl):
            @plsc.parallel_loop(0, n_vec * SCHUNK)
            def body(i):
                s = i & (SCHUNK - 1)
                j = i >> 3
                sli = pl.ds(j * LANES, LANES)
                pv = pos_v[sl][s, sli]
                for b in range(B):
                    buf_v[sl][b, s, sli] = buf_v[sl][b, s, sli] + pv

        ins = {0: start_in(0), 1: start_in(1)}
        outs = {}
        for t in range(n_tiles):
            for h in ins.pop(t):
                h.wait()
            compute(t % NSLOT)
            outs[t] = start_out(t)
            if t + 2 < n_tiles:
                if t >= 1:
                    for h in outs.pop(t - 1):
                        h.wait()
                ins[t + 2] = start_in(t + 2)
        for t in sorted(outs):
            for h in outs.pop(t):
                h.wait()

    return sc_add


def kernel(concat_embeddings, pos_table):
    B, S, H = concat_embeddings.shape
    sc_add = _make_sc_add(B, S, H)
    return sc_add(concat_embeddings, pos_table)


# speedup vs baseline: 1.2818x; 1.0212x over previous
import functools

import jax
import jax.numpy as jnp
from jax import lax
from jax.experimental import pallas as pl
from jax.experimental.pallas import tpu as pltpu
from jax.experimental.pallas import tpu_sc as plsc

NC = 2
NS = 16
NW = NC * NS
LANES = 16
SCHUNK = 8
HCHUNK = 1024
NSLOT = 3


def _make_sc_add(B, S, H):
    pos_per_w = S // NW
    n_sc = pos_per_w // SCHUNK
    n_hc = H // HCHUNK
    n_tiles = n_sc * n_hc
    n_vec = HCHUNK // LANES

    mesh = plsc.VectorSubcoreMesh(core_axis_name="c", subcore_axis_name="s")

    @functools.partial(
        pl.kernel,
        mesh=mesh,
        out_type=jax.ShapeDtypeStruct((B, S, H), jnp.float32),
        scratch_types=(
            [pltpu.VMEM((SCHUNK, HCHUNK), jnp.float32)] * NSLOT
            + [pltpu.VMEM((B, SCHUNK, HCHUNK), jnp.float32)] * NSLOT
            + [pltpu.SemaphoreType.DMA] * (3 * NSLOT)
        ),
        compiler_params=pltpu.CompilerParams(use_tc_tiling_on_sc=True),
    )
    def sc_add(x_hbm, p_hbm, o_hbm, *bufs):
        pos_v = bufs[0:NSLOT]
        buf_v = bufs[NSLOT:2 * NSLOT]
        psem = bufs[2 * NSLOT:3 * NSLOT]
        isem = bufs[3 * NSLOT:4 * NSLOT]
        osem = bufs[4 * NSLOT:5 * NSLOT]
        wid = lax.axis_index("s") * NC + lax.axis_index("c")
        s_base = wid * pos_per_w

        def tile_slices(t):
            c, hi = divmod(t, n_hc)
            s0 = s_base + c * SCHUNK
            return pl.ds(s0, SCHUNK), pl.ds(hi * HCHUNK, HCHUNK)

        def start_in(t, sl):
            ssl, hsl = tile_slices(t)
            pltpu.async_copy(p_hbm.at[ssl, hsl], pos_v[sl], psem[sl])
            pltpu.async_copy(x_hbm.at[:, ssl, hsl], buf_v[sl], isem[sl])

        def start_out(t, sl):
            ssl, hsl = tile_slices(t)
            pltpu.async_copy(buf_v[sl], o_hbm.at[:, ssl, hsl], osem[sl])

        def compute(sl):
            @plsc.parallel_loop(0, n_vec * SCHUNK)
            def body(i):
                s = i & (SCHUNK - 1)
                j = i >> 3
                sli = pl.ds(j * LANES, LANES)
                pv = pos_v[sl][s, sli]
                for b in range(B):
                    buf_v[sl][b, s, sli] = buf_v[sl][b, s, sli] + pv

        def wait_in(t, sl):
            ssl, hsl = tile_slices(t)
            pltpu.make_async_copy(p_hbm.at[ssl, hsl], pos_v[sl], psem[sl]).wait()
            pltpu.make_async_copy(
                x_hbm.at[:, ssl, hsl], buf_v[sl], isem[sl]).wait()

        def wait_out(t, sl):
            ssl, hsl = tile_slices(t)
            pltpu.make_async_copy(
                buf_v[sl], o_hbm.at[:, ssl, hsl], osem[sl]).wait()

        start_in(0, 0)
        start_in(1, 1)
        wait_in(0, 0)
        compute(0)
        start_out(0, 0)
        start_in(2, 2)
        wait_in(1, 1)
        compute(1)
        start_out(1, 1)
        wait_out(0, 0)
        start_in(3, 0)

        def group(g, _):
            t0 = g * 3 + 2
            for u in range(3):
                t = t0 + u
                sl = (2 + u) % NSLOT
                wait_in(t, sl)
                compute(sl)
                start_out(t, sl)
                wait_out(t - 1, (1 + u) % NSLOT)

                @pl.when(t + 2 < n_tiles)
                def _():
                    start_in(t + 2, (1 + u) % NSLOT)
            return 0

        lax.fori_loop(0, (n_tiles - 2) // 3, group, 0)
        wait_out(n_tiles - 1, (n_tiles - 1) % NSLOT)

    return sc_add


def kernel(concat_embeddings, pos_table):
    B, S, H = concat_embeddings.shape
    sc_add = _make_sc_add(B, S, H)
    return sc_add(concat_embeddings, pos_table)
